# Initial kernel scaffold; baseline (speedup 1.0000x reference)
import jax
import jax.numpy as jnp
from jax.experimental import pallas as pl


def kernel(emb_a, rel_ids, emb_b, rel_table, W, b):
    raise NotImplementedError("write your pallas kernel here")


# speedup vs baseline: 1.3560x; 1.3560x over previous
import functools

import jax
import jax.numpy as jnp
from jax import lax
from jax.experimental import pallas as pl
from jax.experimental.pallas import tpu as pltpu
from jax.experimental.pallas import tpu_sc as plsc

E = 800000
EMB_DIM = 64
REL_DIM = 16
N_REL = 1000
N_REL_PAD = 1024
GATHER_W = 128

NC = 2
NS = 16
NW = NC * NS

B_PER_W = E // NW
CHUNK = 128
FULL_CHUNKS = B_PER_W // CHUNK
TAIL = B_PER_W - FULL_CHUNKS * CHUNK


def _project_body(t_ref, w_ref, bias_ref, o_ref):
    wr = w_ref[EMB_DIM:EMB_DIM + REL_DIM, :]
    proj = jnp.dot(t_ref[...], wr, preferred_element_type=jnp.float32)
    proj = proj + bias_ref[...]
    proj = jnp.concatenate(
        [proj, jnp.zeros((N_REL_PAD - N_REL, EMB_DIM), jnp.float32)], axis=0)
    proj = jnp.concatenate(
        [proj, jnp.zeros((N_REL_PAD, GATHER_W - EMB_DIM), jnp.float32)], axis=1)
    o_ref[...] = proj


def _project_table(rel_table, W, bias2d):
    in_dim = 2 * EMB_DIM + REL_DIM
    return pl.pallas_call(
        _project_body,
        in_specs=[
            pl.BlockSpec((N_REL, REL_DIM), lambda: (0, 0)),
            pl.BlockSpec((in_dim, EMB_DIM), lambda: (0, 0)),
            pl.BlockSpec((1, EMB_DIM), lambda: (0, 0)),
        ],
        out_specs=pl.BlockSpec((N_REL_PAD, GATHER_W), lambda: (0, 0)),
        out_shape=jax.ShapeDtypeStruct((N_REL_PAD, GATHER_W), jnp.float32),
    )(rel_table, W, bias2d)


def _sc_gather(table_pad, rel_ids):
    mesh = plsc.VectorSubcoreMesh(core_axis_name="c", subcore_axis_name="s")

    @functools.partial(
        pl.kernel,
        mesh=mesh,
        out_type=jax.ShapeDtypeStruct((E, GATHER_W), jnp.float32),
        scratch_types=[
            pltpu.VMEM((CHUNK,), jnp.int32),
            pltpu.VMEM((CHUNK, GATHER_W), jnp.float32),
            pltpu.VMEM((TAIL,), jnp.int32),
            pltpu.VMEM((TAIL, GATHER_W), jnp.float32),
            pltpu.SemaphoreType.DMA,
        ],
    )
    def k(table_hbm, idx_hbm, out_hbm, idx_v, rows_v, idx_t, rows_t, sem):
        wid = lax.axis_index("s") * NC + lax.axis_index("c")
        base = wid * B_PER_W

        @pl.loop(0, FULL_CHUNKS)
        def _(j):
            off = base + j * CHUNK
            pltpu.sync_copy(idx_hbm.at[pl.ds(off, CHUNK)], idx_v)
            pltpu.async_copy(table_hbm.at[idx_v], rows_v, sem).wait()
            pltpu.sync_copy(rows_v, out_hbm.at[pl.ds(off, CHUNK)])

        off = base + FULL_CHUNKS * CHUNK
        pltpu.sync_copy(idx_hbm.at[pl.ds(off, TAIL)], idx_t)
        pltpu.async_copy(table_hbm.at[idx_t], rows_t, sem).wait()
        pltpu.sync_copy(rows_t, out_hbm.at[pl.ds(off, TAIL)])

    return k(table_pad, rel_ids)


BE = 4000


def _tc_body(a_ref, rel_ref, b2_ref, w_ref, o_ref):
    wa = w_ref[0:EMB_DIM, :]
    wb = w_ref[EMB_DIM + REL_DIM:, :]
    acc = jnp.dot(a_ref[...], wa, preferred_element_type=jnp.float32)
    acc += jnp.dot(b2_ref[...], wb, preferred_element_type=jnp.float32)
    o_ref[...] = acc + rel_ref[:, 0:EMB_DIM]


def _tc_fused(emb_a, rel_g, emb_b, W):
    in_dim = 2 * EMB_DIM + REL_DIM
    grid = (E // BE,)
    return pl.pallas_call(
        _tc_body,
        grid=grid,
        in_specs=[
            pl.BlockSpec((BE, EMB_DIM), lambda i: (i, 0)),
            pl.BlockSpec((BE, GATHER_W), lambda i: (i, 0)),
            pl.BlockSpec((BE, EMB_DIM), lambda i: (i, 0)),
            pl.BlockSpec((in_dim, EMB_DIM), lambda i: (0, 0)),
        ],
        out_specs=pl.BlockSpec((BE, EMB_DIM), lambda i: (i, 0)),
        out_shape=jax.ShapeDtypeStruct((E, EMB_DIM), jnp.float32),
        compiler_params=pltpu.CompilerParams(
            dimension_semantics=("arbitrary",),
        ),
    )(emb_a, rel_g, emb_b, W)


def kernel(emb_a, rel_ids, emb_b, rel_table, W, b):
    table_pad = _project_table(rel_table, W, b.reshape(1, EMB_DIM))
    rel_g = _sc_gather(table_pad, rel_ids)
    return _tc_fused(emb_a, rel_g, emb_b, W)
